# baseline (speedup 1.0000x reference)
import jax
import jax.numpy as jnp
from jax import lax
from jax.experimental import pallas as pl
from jax.experimental.pallas import tpu as pltpu
from jax.experimental.pallas import tpu_sc as plsc

_N = 10000
_D = 128
_DH = _D // 2
_NC = 2
_NS = 16
_CHUNK = 128
_N_PAD = 10112
_RPT = _N_PAD // _NS

_SC_PARAMS = pltpu.CompilerParams(use_tc_tiling_on_sc=False)


def _make_mp(cpt):
    mesh = plsc.VectorSubcoreMesh(core_axis_name="c", subcore_axis_name="s")
    nbuf = 6
    pref = 3
    assert cpt >= nbuf
    out_type = [jax.ShapeDtypeStruct((_N_PAD, _D), jnp.float32)]
    scratch = [
        pltpu.VMEM((cpt, _CHUNK), jnp.int32),
        pltpu.VMEM((cpt, _CHUNK), jnp.int32),
        [pltpu.VMEM((_CHUNK, _DH), jnp.float32)] * nbuf,
        pltpu.VMEM_SHARED((_N_PAD, _DH), jnp.float32),
        [pltpu.SemaphoreType.DMA] * nbuf,
        [pltpu.SemaphoreType.DMA] * nbuf,
    ]

    def body(h2_hbm, src_hbm, dst_hbm, zrow_hbm,
             agg_out, src_v, dst_v, bufs, agg_sh, gsem, ssem):
        c = lax.axis_index("c")
        s = lax.axis_index("s")
        pltpu.sync_copy(src_hbm.at[pl.ds(s * cpt, cpt)], src_v)
        pltpu.sync_copy(dst_hbm.at[pl.ds(s * cpt, cpt)], dst_v)
        pltpu.sync_copy(zrow_hbm, agg_sh.at[pl.ds(s * _RPT, _RPT)])
        @pl.when(c == 1)
        def _():
            def fix(j, _):
                for k in range(_CHUNK // 16):
                    src_v[j, pl.ds(k * 16, 16)] = (
                        src_v[j, pl.ds(k * 16, 16)] + 1)
                return 0
            lax.fori_loop(0, cpt, fix, 0)
        plsc.subcore_barrier()

        for k in range(pref):
            pltpu.async_copy(h2_hbm.at[src_v.at[k]], bufs[k], gsem[k])

        def step(j, _):
            for b in range(nbuf):
                @pl.when(lax.rem(j, nbuf) == b)
                def _():
                    pltpu.make_async_copy(
                        h2_hbm.at[src_v.at[j]], bufs[b], gsem[b]).wait()
                    pltpu.async_copy(bufs[b], agg_sh.at[dst_v.at[j]],
                                     ssem[b], add=True)
                    bp = (b + pref) % nbuf

                    @pl.when(j + pref < cpt)
                    def _():
                        @pl.when(j + pref >= nbuf)
                        def _():
                            pltpu.make_async_copy(
                                bufs[bp], agg_sh.at[dst_v.at[0]],
                                ssem[bp]).wait()
                        pltpu.async_copy(h2_hbm.at[src_v.at[j + pref]],
                                         bufs[bp], gsem[bp])
            return 0

        lax.fori_loop(0, cpt, step, 0)
        for b in range(nbuf):
            pltpu.make_async_copy(bufs[b], agg_sh.at[dst_v.at[0]],
                                  ssem[b]).wait()
        plsc.subcore_barrier()
        pltpu.sync_copy(agg_sh.at[pl.ds(s * _RPT, _RPT)],
                        agg_out.at[pl.ds(s * _RPT, _RPT),
                                   pl.ds(c * _DH, _DH)])

    return pl.kernel(body, out_type=out_type, mesh=mesh,
                     scratch_types=scratch, compiler_params=_SC_PARAMS)


def _make_deg(cpt):
    mesh = plsc.VectorSubcoreMesh(core_axis_name="c", subcore_axis_name="s")
    out_type = [jax.ShapeDtypeStruct((_NC, _N_PAD, 16), jnp.float32)]
    scratch = [
        pltpu.VMEM((cpt, _CHUNK), jnp.int32),
        pltpu.VMEM((_CHUNK, 16), jnp.float32),
        pltpu.VMEM_SHARED((_N_PAD, 16), jnp.float32),
        [pltpu.SemaphoreType.DMA] * 2,
    ]
    half = -(-cpt // 2)

    def body(dst_hbm, zcol_hbm, ones_hbm, deg_out, dst_v, ones_v, deg_sh,
             dsem):
        c = lax.axis_index("c")
        s = lax.axis_index("s")
        pltpu.sync_copy(dst_hbm.at[pl.ds(s * cpt, cpt)], dst_v)
        pltpu.sync_copy(ones_hbm, ones_v)
        pltpu.sync_copy(zcol_hbm, deg_sh.at[pl.ds(s * _RPT, _RPT)])
        plsc.subcore_barrier()

        def step(i, _):
            j = 2 * i + c
            for p in range(2):
                @pl.when(lax.rem(i, 2) == p)
                def _():
                    @pl.when(j < cpt)
                    def _():
                        @pl.when(i >= 2)
                        def _():
                            pltpu.make_async_copy(
                                ones_v, deg_sh.at[dst_v.at[0]],
                                dsem[p]).wait()
                        pltpu.async_copy(ones_v, deg_sh.at[dst_v.at[j]],
                                         dsem[p], add=True)
            return 0

        lax.fori_loop(0, half, step, 0)
        for p in range(2):
            pltpu.make_async_copy(ones_v, deg_sh.at[dst_v.at[0]],
                                  dsem[p]).wait()
        plsc.subcore_barrier()
        pltpu.sync_copy(deg_sh.at[pl.ds(s * _RPT, _RPT)],
                        deg_out.at[c, pl.ds(s * _RPT, _RPT)])

    return pl.kernel(body, out_type=out_type, mesh=mesh,
                     scratch_types=scratch, compiler_params=_SC_PARAMS)


def _tc_layer(first, last):
    def body(agg_ref, deg_ref, w_ref, gamma_ref, beta_ref, alpha_ref,
             out_ref, *maybe_degc):
        agg = agg_ref[:_N, :]
        if first:
            deg = jnp.maximum(deg_ref[0, :_N, 0:1] + deg_ref[1, :_N, 0:1],
                              1.0)
            maybe_degc[0][...] = deg
        else:
            deg = deg_ref[...]
        t = agg / deg
        g = jnp.dot(t, w_ref[...], preferred_element_type=jnp.float32)
        mean = jnp.mean(g, axis=0, keepdims=True)
        shifted = g - alpha_ref[...] * mean
        var = jnp.mean(shifted * shifted, axis=0, keepdims=True)
        h = shifted * lax.rsqrt(var + 1e-5) * gamma_ref[...] + beta_ref[...]
        h = jnp.maximum(h, 0.0)
        if last:
            nrm = jnp.sqrt(jnp.sum(h * h, axis=1, keepdims=True))
            h = h / jnp.maximum(nrm, 1e-12)
        out_ref[...] = h

    out_shape = [jax.ShapeDtypeStruct((_N, _D), jnp.float32)]
    if first:
        out_shape.append(jax.ShapeDtypeStruct((_N, 1), jnp.float32))
    return pl.pallas_call(body, out_shape=out_shape)


def kernel(x, edge_index, Ws, gammas, betas, alphas):
    e = edge_index.shape[1]
    cpt = -(-e // (_NS * _CHUNK))
    e_pad = _NS * cpt * _CHUNK
    src = jnp.concatenate(
        [edge_index[0] * 2, jnp.zeros((e_pad - e,), jnp.int32)]
    ).reshape(_NS * cpt, _CHUNK)
    dst = jnp.concatenate(
        [edge_index[1], jnp.full((e_pad - e,), _N, jnp.int32)]
    ).reshape(_NS * cpt, _CHUNK)
    zrow = jnp.zeros((_RPT, _DH), jnp.float32)
    zcol = jnp.zeros((_RPT, 16), jnp.float32)
    ones = jnp.ones((_CHUNK, 16), jnp.float32)

    mp = _make_mp(cpt)

    (degp,) = _make_deg(cpt)(dst, zcol, ones)
    (agg,) = mp(x.reshape(2 * _N, _DH), src, dst, zrow)
    h, degc = _tc_layer(True, False)(
        agg, degp, Ws[0], gammas[0][None, :], betas[0][None, :],
        alphas[0][None, :])
    (agg,) = mp(h.reshape(2 * _N, _DH), src, dst, zrow)
    (h,) = _tc_layer(False, False)(
        agg, degc, Ws[1], gammas[1][None, :], betas[1][None, :],
        alphas[1][None, :])
    (agg,) = mp(h.reshape(2 * _N, _DH), src, dst, zrow)
    (h,) = _tc_layer(False, True)(
        agg, degc, Ws[2], gammas[2][None, :], betas[2][None, :],
        alphas[2][None, :])
    return h


# speedup vs baseline: 1.0680x; 1.0680x over previous
import jax
import jax.numpy as jnp
from jax import lax
from jax.experimental import pallas as pl
from jax.experimental.pallas import tpu as pltpu
from jax.experimental.pallas import tpu_sc as plsc

_N = 10000
_D = 128
_DH = _D // 2
_NC = 2
_NS = 16
_CHUNK = 128
_N_PAD = 10112
_RPT = _N_PAD // _NS

_SC_PARAMS = pltpu.CompilerParams(use_tc_tiling_on_sc=False)


def _make_mp(cpt):
    mesh = plsc.VectorSubcoreMesh(core_axis_name="c", subcore_axis_name="s")
    nbuf = 6
    pref = 4
    assert cpt >= nbuf
    out_type = [jax.ShapeDtypeStruct((_N_PAD, _D), jnp.float32)]
    scratch = [
        pltpu.VMEM((cpt, _CHUNK), jnp.int32),
        pltpu.VMEM((cpt, _CHUNK), jnp.int32),
        [pltpu.VMEM((_CHUNK, _DH), jnp.float32)] * nbuf,
        pltpu.VMEM_SHARED((_N_PAD, _DH), jnp.float32),
        [pltpu.SemaphoreType.DMA] * nbuf,
        [pltpu.SemaphoreType.DMA] * nbuf,
    ]

    def body(h2_hbm, src_hbm, dst_hbm, zrow_hbm,
             agg_out, src_v, dst_v, bufs, agg_sh, gsem, ssem):
        c = lax.axis_index("c")
        s = lax.axis_index("s")
        pltpu.sync_copy(src_hbm.at[pl.ds(s * cpt, cpt)], src_v)
        pltpu.sync_copy(dst_hbm.at[pl.ds(s * cpt, cpt)], dst_v)
        pltpu.sync_copy(zrow_hbm, agg_sh.at[pl.ds(s * _RPT, _RPT)])
        @pl.when(c == 1)
        def _():
            def fix(j, _):
                for k in range(_CHUNK // 16):
                    src_v[j, pl.ds(k * 16, 16)] = (
                        src_v[j, pl.ds(k * 16, 16)] + 1)
                return 0
            lax.fori_loop(0, cpt, fix, 0)
        plsc.subcore_barrier()

        for k in range(pref):
            pltpu.async_copy(h2_hbm.at[src_v.at[k]], bufs[k], gsem[k])

        def step(j, _):
            for b in range(nbuf):
                @pl.when(lax.rem(j, nbuf) == b)
                def _():
                    pltpu.make_async_copy(
                        h2_hbm.at[src_v.at[j]], bufs[b], gsem[b]).wait()
                    pltpu.async_copy(bufs[b], agg_sh.at[dst_v.at[j]],
                                     ssem[b], add=True)
                    bp = (b + pref) % nbuf

                    @pl.when(j + pref < cpt)
                    def _():
                        @pl.when(j + pref >= nbuf)
                        def _():
                            pltpu.make_async_copy(
                                bufs[bp], agg_sh.at[dst_v.at[0]],
                                ssem[bp]).wait()
                        pltpu.async_copy(h2_hbm.at[src_v.at[j + pref]],
                                         bufs[bp], gsem[bp])
            return 0

        lax.fori_loop(0, cpt, step, 0)
        for b in range(nbuf):
            pltpu.make_async_copy(bufs[b], agg_sh.at[dst_v.at[0]],
                                  ssem[b]).wait()
        plsc.subcore_barrier()
        pltpu.sync_copy(agg_sh.at[pl.ds(s * _RPT, _RPT)],
                        agg_out.at[pl.ds(s * _RPT, _RPT),
                                   pl.ds(c * _DH, _DH)])

    return pl.kernel(body, out_type=out_type, mesh=mesh,
                     scratch_types=scratch, compiler_params=_SC_PARAMS)


def _make_deg(cpt):
    mesh = plsc.VectorSubcoreMesh(core_axis_name="c", subcore_axis_name="s")
    out_type = [jax.ShapeDtypeStruct((_NC, _N_PAD, 16), jnp.float32)]
    scratch = [
        pltpu.VMEM((cpt, _CHUNK), jnp.int32),
        pltpu.VMEM((_CHUNK, 16), jnp.float32),
        pltpu.VMEM_SHARED((_N_PAD, 16), jnp.float32),
        [pltpu.SemaphoreType.DMA] * 2,
    ]
    half = -(-cpt // 2)

    def body(dst_hbm, zcol_hbm, ones_hbm, deg_out, dst_v, ones_v, deg_sh,
             dsem):
        c = lax.axis_index("c")
        s = lax.axis_index("s")
        pltpu.sync_copy(dst_hbm.at[pl.ds(s * cpt, cpt)], dst_v)
        pltpu.sync_copy(ones_hbm, ones_v)
        pltpu.sync_copy(zcol_hbm, deg_sh.at[pl.ds(s * _RPT, _RPT)])
        plsc.subcore_barrier()

        def step(i, _):
            j = 2 * i + c
            for p in range(2):
                @pl.when(lax.rem(i, 2) == p)
                def _():
                    @pl.when(j < cpt)
                    def _():
                        @pl.when(i >= 2)
                        def _():
                            pltpu.make_async_copy(
                                ones_v, deg_sh.at[dst_v.at[0]],
                                dsem[p]).wait()
                        pltpu.async_copy(ones_v, deg_sh.at[dst_v.at[j]],
                                         dsem[p], add=True)
            return 0

        lax.fori_loop(0, half, step, 0)
        for p in range(2):
            pltpu.make_async_copy(ones_v, deg_sh.at[dst_v.at[0]],
                                  dsem[p]).wait()
        plsc.subcore_barrier()
        pltpu.sync_copy(deg_sh.at[pl.ds(s * _RPT, _RPT)],
                        deg_out.at[c, pl.ds(s * _RPT, _RPT)])

    return pl.kernel(body, out_type=out_type, mesh=mesh,
                     scratch_types=scratch, compiler_params=_SC_PARAMS)


def _tc_layer(first, last):
    def body(agg_ref, deg_ref, w_ref, gamma_ref, beta_ref, alpha_ref,
             out_ref, *maybe_degc):
        agg = agg_ref[:_N, :]
        if first:
            deg = jnp.maximum(deg_ref[0, :_N, 0:1] + deg_ref[1, :_N, 0:1],
                              1.0)
            maybe_degc[0][...] = deg
        else:
            deg = deg_ref[...]
        t = agg / deg
        g = jnp.dot(t, w_ref[...], preferred_element_type=jnp.float32)
        mean = jnp.mean(g, axis=0, keepdims=True)
        shifted = g - alpha_ref[...] * mean
        var = jnp.mean(shifted * shifted, axis=0, keepdims=True)
        h = shifted * lax.rsqrt(var + 1e-5) * gamma_ref[...] + beta_ref[...]
        h = jnp.maximum(h, 0.0)
        if last:
            nrm = jnp.sqrt(jnp.sum(h * h, axis=1, keepdims=True))
            h = h / jnp.maximum(nrm, 1e-12)
        out_ref[...] = h

    out_shape = [jax.ShapeDtypeStruct((_N, _D), jnp.float32)]
    if first:
        out_shape.append(jax.ShapeDtypeStruct((_N, 1), jnp.float32))
    return pl.pallas_call(body, out_shape=out_shape)


def kernel(x, edge_index, Ws, gammas, betas, alphas):
    e = edge_index.shape[1]
    cpt = -(-e // (_NS * _CHUNK))
    e_pad = _NS * cpt * _CHUNK
    src = jnp.concatenate(
        [edge_index[0] * 2, jnp.zeros((e_pad - e,), jnp.int32)]
    ).reshape(_NS * cpt, _CHUNK)
    dst = jnp.concatenate(
        [edge_index[1], jnp.full((e_pad - e,), _N, jnp.int32)]
    ).reshape(_NS * cpt, _CHUNK)
    zrow = jnp.zeros((_RPT, _DH), jnp.float32)
    zcol = jnp.zeros((_RPT, 16), jnp.float32)
    ones = jnp.ones((_CHUNK, 16), jnp.float32)

    mp = _make_mp(cpt)

    (degp,) = _make_deg(cpt)(dst, zcol, ones)
    (agg,) = mp(x.reshape(2 * _N, _DH), src, dst, zrow)
    h, degc = _tc_layer(True, False)(
        agg, degp, Ws[0], gammas[0][None, :], betas[0][None, :],
        alphas[0][None, :])
    (agg,) = mp(h.reshape(2 * _N, _DH), src, dst, zrow)
    (h,) = _tc_layer(False, False)(
        agg, degc, Ws[1], gammas[1][None, :], betas[1][None, :],
        alphas[1][None, :])
    (agg,) = mp(h.reshape(2 * _N, _DH), src, dst, zrow)
    (h,) = _tc_layer(False, True)(
        agg, degc, Ws[2], gammas[2][None, :], betas[2][None, :],
        alphas[2][None, :])
    return h


# speedup vs baseline: 1.0683x; 1.0004x over previous
import jax
import jax.numpy as jnp
from jax import lax
from jax.experimental import pallas as pl
from jax.experimental.pallas import tpu as pltpu
from jax.experimental.pallas import tpu_sc as plsc

_N = 10000
_D = 128
_DH = _D // 2
_NC = 2
_NS = 16
_CHUNK = 128
_N_PAD = 10112
_RPT = _N_PAD // _NS

_SC_PARAMS = pltpu.CompilerParams(use_tc_tiling_on_sc=False)


def _make_mp(cpt):
    mesh = plsc.VectorSubcoreMesh(core_axis_name="c", subcore_axis_name="s")
    nbuf = 6
    pref = 5
    assert cpt >= nbuf
    out_type = [jax.ShapeDtypeStruct((_N_PAD, _D), jnp.float32)]
    scratch = [
        pltpu.VMEM((cpt, _CHUNK), jnp.int32),
        pltpu.VMEM((cpt, _CHUNK), jnp.int32),
        [pltpu.VMEM((_CHUNK, _DH), jnp.float32)] * nbuf,
        pltpu.VMEM_SHARED((_N_PAD, _DH), jnp.float32),
        [pltpu.SemaphoreType.DMA] * nbuf,
        [pltpu.SemaphoreType.DMA] * nbuf,
    ]

    def body(h2_hbm, src_hbm, dst_hbm, zrow_hbm,
             agg_out, src_v, dst_v, bufs, agg_sh, gsem, ssem):
        c = lax.axis_index("c")
        s = lax.axis_index("s")
        pltpu.sync_copy(src_hbm.at[pl.ds(s * cpt, cpt)], src_v)
        pltpu.sync_copy(dst_hbm.at[pl.ds(s * cpt, cpt)], dst_v)
        pltpu.sync_copy(zrow_hbm, agg_sh.at[pl.ds(s * _RPT, _RPT)])
        @pl.when(c == 1)
        def _():
            def fix(j, _):
                for k in range(_CHUNK // 16):
                    src_v[j, pl.ds(k * 16, 16)] = (
                        src_v[j, pl.ds(k * 16, 16)] + 1)
                return 0
            lax.fori_loop(0, cpt, fix, 0)
        plsc.subcore_barrier()

        for k in range(pref):
            pltpu.async_copy(h2_hbm.at[src_v.at[k]], bufs[k], gsem[k])

        def step(j, _):
            for b in range(nbuf):
                @pl.when(lax.rem(j, nbuf) == b)
                def _():
                    pltpu.make_async_copy(
                        h2_hbm.at[src_v.at[j]], bufs[b], gsem[b]).wait()
                    pltpu.async_copy(bufs[b], agg_sh.at[dst_v.at[j]],
                                     ssem[b], add=True)
                    bp = (b + pref) % nbuf

                    @pl.when(j + pref < cpt)
                    def _():
                        @pl.when(j + pref >= nbuf)
                        def _():
                            pltpu.make_async_copy(
                                bufs[bp], agg_sh.at[dst_v.at[0]],
                                ssem[bp]).wait()
                        pltpu.async_copy(h2_hbm.at[src_v.at[j + pref]],
                                         bufs[bp], gsem[bp])
            return 0

        lax.fori_loop(0, cpt, step, 0)
        for b in range(nbuf):
            pltpu.make_async_copy(bufs[b], agg_sh.at[dst_v.at[0]],
                                  ssem[b]).wait()
        plsc.subcore_barrier()
        pltpu.sync_copy(agg_sh.at[pl.ds(s * _RPT, _RPT)],
                        agg_out.at[pl.ds(s * _RPT, _RPT),
                                   pl.ds(c * _DH, _DH)])

    return pl.kernel(body, out_type=out_type, mesh=mesh,
                     scratch_types=scratch, compiler_params=_SC_PARAMS)


def _make_deg(cpt):
    mesh = plsc.VectorSubcoreMesh(core_axis_name="c", subcore_axis_name="s")
    out_type = [jax.ShapeDtypeStruct((_NC, _N_PAD, 16), jnp.float32)]
    scratch = [
        pltpu.VMEM((cpt, _CHUNK), jnp.int32),
        pltpu.VMEM((_CHUNK, 16), jnp.float32),
        pltpu.VMEM_SHARED((_N_PAD, 16), jnp.float32),
        [pltpu.SemaphoreType.DMA] * 2,
    ]
    half = -(-cpt // 2)

    def body(dst_hbm, zcol_hbm, ones_hbm, deg_out, dst_v, ones_v, deg_sh,
             dsem):
        c = lax.axis_index("c")
        s = lax.axis_index("s")
        pltpu.sync_copy(dst_hbm.at[pl.ds(s * cpt, cpt)], dst_v)
        pltpu.sync_copy(ones_hbm, ones_v)
        pltpu.sync_copy(zcol_hbm, deg_sh.at[pl.ds(s * _RPT, _RPT)])
        plsc.subcore_barrier()

        def step(i, _):
            j = 2 * i + c
            for p in range(2):
                @pl.when(lax.rem(i, 2) == p)
                def _():
                    @pl.when(j < cpt)
                    def _():
                        @pl.when(i >= 2)
                        def _():
                            pltpu.make_async_copy(
                                ones_v, deg_sh.at[dst_v.at[0]],
                                dsem[p]).wait()
                        pltpu.async_copy(ones_v, deg_sh.at[dst_v.at[j]],
                                         dsem[p], add=True)
            return 0

        lax.fori_loop(0, half, step, 0)
        for p in range(2):
            pltpu.make_async_copy(ones_v, deg_sh.at[dst_v.at[0]],
                                  dsem[p]).wait()
        plsc.subcore_barrier()
        pltpu.sync_copy(deg_sh.at[pl.ds(s * _RPT, _RPT)],
                        deg_out.at[c, pl.ds(s * _RPT, _RPT)])

    return pl.kernel(body, out_type=out_type, mesh=mesh,
                     scratch_types=scratch, compiler_params=_SC_PARAMS)


def _tc_layer(first, last):
    def body(agg_ref, deg_ref, w_ref, gamma_ref, beta_ref, alpha_ref,
             out_ref, *maybe_degc):
        agg = agg_ref[:_N, :]
        if first:
            deg = jnp.maximum(deg_ref[0, :_N, 0:1] + deg_ref[1, :_N, 0:1],
                              1.0)
            maybe_degc[0][...] = deg
        else:
            deg = deg_ref[...]
        t = agg / deg
        g = jnp.dot(t, w_ref[...], preferred_element_type=jnp.float32)
        mean = jnp.mean(g, axis=0, keepdims=True)
        shifted = g - alpha_ref[...] * mean
        var = jnp.mean(shifted * shifted, axis=0, keepdims=True)
        h = shifted * lax.rsqrt(var + 1e-5) * gamma_ref[...] + beta_ref[...]
        h = jnp.maximum(h, 0.0)
        if last:
            nrm = jnp.sqrt(jnp.sum(h * h, axis=1, keepdims=True))
            h = h / jnp.maximum(nrm, 1e-12)
        out_ref[...] = h

    out_shape = [jax.ShapeDtypeStruct((_N, _D), jnp.float32)]
    if first:
        out_shape.append(jax.ShapeDtypeStruct((_N, 1), jnp.float32))
    return pl.pallas_call(body, out_shape=out_shape)


def kernel(x, edge_index, Ws, gammas, betas, alphas):
    e = edge_index.shape[1]
    cpt = -(-e // (_NS * _CHUNK))
    e_pad = _NS * cpt * _CHUNK
    src = jnp.concatenate(
        [edge_index[0] * 2, jnp.zeros((e_pad - e,), jnp.int32)]
    ).reshape(_NS * cpt, _CHUNK)
    dst = jnp.concatenate(
        [edge_index[1], jnp.full((e_pad - e,), _N, jnp.int32)]
    ).reshape(_NS * cpt, _CHUNK)
    zrow = jnp.zeros((_RPT, _DH), jnp.float32)
    zcol = jnp.zeros((_RPT, 16), jnp.float32)
    ones = jnp.ones((_CHUNK, 16), jnp.float32)

    mp = _make_mp(cpt)

    (degp,) = _make_deg(cpt)(dst, zcol, ones)
    (agg,) = mp(x.reshape(2 * _N, _DH), src, dst, zrow)
    h, degc = _tc_layer(True, False)(
        agg, degp, Ws[0], gammas[0][None, :], betas[0][None, :],
        alphas[0][None, :])
    (agg,) = mp(h.reshape(2 * _N, _DH), src, dst, zrow)
    (h,) = _tc_layer(False, False)(
        agg, degc, Ws[1], gammas[1][None, :], betas[1][None, :],
        alphas[1][None, :])
    (agg,) = mp(h.reshape(2 * _N, _DH), src, dst, zrow)
    (h,) = _tc_layer(False, True)(
        agg, degc, Ws[2], gammas[2][None, :], betas[2][None, :],
        alphas[2][None, :])
    return h
